# baseline (speedup 1.0000x reference)
import dataclasses
import functools

import jax
import jax.numpy as jnp
from jax.experimental import pallas as pl
from jax.experimental.pallas import tpu as pltpu
from jax.experimental.pallas import tpu_sc as plsc

_B, _H, _W, _A, _NC = 8, 64, 64, 3, 3
_SIZES = (16.0, 32.0, 64.0)
_STRIDE = 8.0
_G = 20
_ROWS = (_H * _W) // 128
_NA = _H * _W * _A
_NT = 6


def _smooth_l1(x, t):
    d = jnp.abs(x - t)
    return jnp.where(d < 1.0, 0.5 * d * d, d - 0.5)


def _anchor_centers():
    col = jax.lax.broadcasted_iota(jnp.int32, (_ROWS, 128), 1)
    row = jax.lax.broadcasted_iota(jnp.int32, (_ROWS, 128), 0)
    x = jnp.remainder(col, _W)
    y = 2 * row + col // _W
    ax = (x.astype(jnp.float32) + 0.5) * _STRIDE
    ay = (y.astype(jnp.float32) + 0.5) * _STRIDE
    return ax, ay, col


def _match_kernel(pred_ref, gtb_ref, gtl_ref, out_ref, neg_ref, tgt_ref):
    b = pl.program_id(0)
    ax, ay, lane = _anchor_centers()
    eps = jnp.float32(1e-6)

    objp_sum = jnp.float32(0.0)
    pos_cnt = jnp.float32(0.0)
    neg_cnt = jnp.float32(0.0)

    for a in range(_A):
        s = _SIZES[a]
        half = s * 0.5
        inv_s = 1.0 / s
        area_a = s * s
        ax1, ay1, ax2, ay2 = ax - half, ay - half, ax + half, ay + half

        best = jnp.full((_ROWS, 128), -1.0, dtype=jnp.float32)
        m_cx = jnp.zeros((_ROWS, 128), dtype=jnp.float32)
        m_cy = jnp.zeros((_ROWS, 128), dtype=jnp.float32)
        m_w = jnp.ones((_ROWS, 128), dtype=jnp.float32)
        m_h = jnp.ones((_ROWS, 128), dtype=jnp.float32)
        m_lab = jnp.zeros((_ROWS, 128), dtype=jnp.float32)

        for g in range(_G):
            gx1 = gtb_ref[b, g, 0]
            gy1 = gtb_ref[b, g, 1]
            gx2 = gtb_ref[b, g, 2]
            gy2 = gtb_ref[b, g, 3]
            glab = gtl_ref[b, g].astype(jnp.float32)
            ix1 = jnp.maximum(ax1, gx1)
            iy1 = jnp.maximum(ay1, gy1)
            ix2 = jnp.minimum(ax2, gx2)
            iy2 = jnp.minimum(ay2, gy2)
            inter = jnp.clip(ix2 - ix1, 0.0) * jnp.clip(iy2 - iy1, 0.0)
            area_g = (gx2 - gx1) * (gy2 - gy1)
            union = area_a + area_g - inter
            iou = inter / jnp.maximum(union, 1e-9)
            upd = iou > best
            best = jnp.where(upd, iou, best)
            m_cx = jnp.where(upd, (gx1 + gx2) * 0.5, m_cx)
            m_cy = jnp.where(upd, (gy1 + gy2) * 0.5, m_cy)
            m_w = jnp.where(upd, jnp.maximum(gx2 - gx1, eps), m_w)
            m_h = jnp.where(upd, jnp.maximum(gy2 - gy1, eps), m_h)
            m_lab = jnp.where(upd, glab, m_lab)

        posf = (best >= 0.5).astype(jnp.float32)
        negm = best < 0.4
        pos_cnt += jnp.sum(posf)
        neg_cnt += jnp.sum(negm.astype(jnp.float32))

        p_obj = pred_ref[0, a * (5 + _NC) + 4, :, :]
        bce = (
            jnp.maximum(p_obj, 0.0)
            - p_obj * posf
            + jnp.log1p(jnp.exp(-jnp.abs(p_obj)))
        )
        objp_sum += jnp.sum(bce * posf)
        neg_ref[0, a * _ROWS : (a + 1) * _ROWS, :] = jnp.where(negm, bce, 0.0)

        t0 = a * _NT * _ROWS
        tgt_ref[0, t0 : t0 + _ROWS, :] = (m_cx - ax) * inv_s
        tgt_ref[0, t0 + _ROWS : t0 + 2 * _ROWS, :] = (m_cy - ay) * inv_s
        tgt_ref[0, t0 + 2 * _ROWS : t0 + 3 * _ROWS, :] = jnp.log(m_w * inv_s)
        tgt_ref[0, t0 + 3 * _ROWS : t0 + 4 * _ROWS, :] = jnp.log(m_h * inv_s)
        tgt_ref[0, t0 + 4 * _ROWS : t0 + 5 * _ROWS, :] = posf
        tgt_ref[0, t0 + 5 * _ROWS : t0 + 6 * _ROWS, :] = m_lab

    k_f = jnp.minimum(
        3.0 * jnp.maximum(1.0, pos_cnt), neg_cnt
    )
    out_row = (
        jnp.where(lane[:1, :] == 1, objp_sum, 0.0)
        + jnp.where(lane[:1, :] == 3, pos_cnt, 0.0)
        + jnp.where(lane[:1, :] == 4, neg_cnt, 0.0)
        + jnp.where(lane[:1, :] == 5, k_f, 0.0)
    )
    out_ref[0, :, :] = out_row


def _loss2_kernel(pred_ref, tgt_ref, out_ref):
    lane = jax.lax.broadcasted_iota(jnp.int32, (1, 128), 1)
    loc_sum = jnp.float32(0.0)
    cls_sum = jnp.float32(0.0)

    for a in range(_A):
        base = a * (5 + _NC)
        t0 = a * _NT * _ROWS
        t_tx = tgt_ref[0, t0 : t0 + _ROWS, :]
        t_ty = tgt_ref[0, t0 + _ROWS : t0 + 2 * _ROWS, :]
        t_tw = tgt_ref[0, t0 + 2 * _ROWS : t0 + 3 * _ROWS, :]
        t_th = tgt_ref[0, t0 + 3 * _ROWS : t0 + 4 * _ROWS, :]
        posf = tgt_ref[0, t0 + 4 * _ROWS : t0 + 5 * _ROWS, :]
        m_lab = tgt_ref[0, t0 + 5 * _ROWS : t0 + 6 * _ROWS, :]

        loc_plane = (
            _smooth_l1(pred_ref[0, base + 0, :, :], t_tx)
            + _smooth_l1(pred_ref[0, base + 1, :, :], t_ty)
            + _smooth_l1(pred_ref[0, base + 2, :, :], t_tw)
            + _smooth_l1(pred_ref[0, base + 3, :, :], t_th)
        )
        loc_sum += jnp.sum(loc_plane * posf)

        c0 = pred_ref[0, base + 5, :, :]
        c1 = pred_ref[0, base + 6, :, :]
        c2 = pred_ref[0, base + 7, :, :]
        m = jnp.maximum(jnp.maximum(c0, c1), c2)
        lse = m + jnp.log(
            jnp.exp(c0 - m) + jnp.exp(c1 - m) + jnp.exp(c2 - m)
        )
        picked = jnp.where(m_lab < 0.5, c0, jnp.where(m_lab < 1.5, c1, c2))
        cls_sum += jnp.sum((lse - picked) * posf)

    out_row = jnp.where(lane == 0, loc_sum, 0.0) + jnp.where(
        lane == 2, cls_sum, 0.0
    )
    out_ref[0, :, :] = out_row


_NSLOT = 2


def _sc_mine(neg_flat, sums_rows, core_sel, nb):
    mesh = plsc.VectorSubcoreMesh(core_axis_name="c", subcore_axis_name="s")
    cp = pltpu.CompilerParams()
    if "needs_layout_passes" in pltpu.CompilerParams.__dataclass_fields__:
        cp = dataclasses.replace(cp, needs_layout_passes=False)
    hsize = _NSLOT * 16 * 256

    @functools.partial(
        pl.kernel,
        mesh=mesh,
        compiler_params=cp,
        out_type=jax.ShapeDtypeStruct((nb, 16), jnp.float32),
        scratch_types=[
            pltpu.VMEM((_NA,), jnp.float32),
            pltpu.VMEM((hsize,), jnp.int32),
            pltpu.VMEM((hsize,), jnp.float32),
            pltpu.VMEM((1, 128), jnp.float32),
            pltpu.VMEM((16,), jnp.float32),
            pltpu.SemaphoreType.DMA,
        ],
    )
    def mine(neg_hbm, row_hbm, out_hbm, data, hist, sums, krow, outv, sem):
        sid = jax.lax.axis_index("s")
        cid = jax.lax.axis_index("c")

        @pl.when((cid == core_sel) & (sid < nb))
        def _():
            b = sid
            pltpu.async_copy(neg_hbm.at[b], data, sem).wait()
            pltpu.async_copy(row_hbm.at[b], krow, sem).wait()
            lane = jax.lax.iota(jnp.int32, 16)
            lane256 = lane * 256
            ones = jnp.full((16,), 1, jnp.int32)
            zero16i = jnp.zeros((16,), jnp.int32)
            zero16f = jnp.zeros((16,), jnp.float32)
            k_splat = plsc.load_gather(
                krow, [zero16i, jnp.full((16,), 5, jnp.int32)]
            ).astype(jnp.int32)

            def splat_i(v):
                return jnp.full((16,), v, jnp.int32)

            def splat_f(v):
                return jnp.full((16,), v, jnp.float32)

            def run_pass(shift, k_s, prev_shift, prev_beta):
                @plsc.parallel_loop(0, hsize, step=16)
                def _(i):
                    hist[pl.ds(i, 16)] = zero16i
                    sums[pl.ds(i, 16)] = zero16f

                @plsc.parallel_loop(0, _NA, step=16, unroll=_NSLOT)
                def _(i):
                    v = data[pl.ds(i, 16)]
                    bits = plsc.bitcast(v, jnp.int32)
                    d = jnp.bitwise_and(
                        jax.lax.shift_right_logical(bits, shift), 255
                    )
                    slot = jnp.bitwise_and(
                        jax.lax.shift_right_logical(i, 4), _NSLOT - 1
                    )
                    idx = slot * 4096 + lane256 + d
                    if prev_shift is None:
                        plsc.addupdate_scatter(hist, [idx], ones)
                        plsc.addupdate_scatter(sums, [idx], v)
                    else:
                        pd = jnp.bitwise_and(
                            jax.lax.shift_right_logical(bits, prev_shift), 255
                        )
                        m = pd == prev_beta
                        plsc.addupdate_scatter(hist, [idx], ones, mask=m)
                        plsc.addupdate_scatter(sums, [idx], v, mask=m)

                cum = splat_i(0)
                cums = splat_f(0.0)
                beta_v = splat_i(0)
                cnta_v = splat_i(0)
                suma_v = splat_f(0.0)
                for j in reversed(range(16)):
                    hv = hist[pl.ds(j * 16, 16)]
                    sv = sums[pl.ds(j * 16, 16)]
                    for t in range(1, 16 * _NSLOT):
                        hv = hv + hist[pl.ds(t * 256 + j * 16, 16)]
                        sv = sv + sums[pl.ds(t * 256 + j * 16, 16)]
                    sfx_h = jax.lax.rev(plsc.cumsum(jax.lax.rev(hv, (0,))), (0,))
                    sfx_s = jax.lax.rev(plsc.cumsum(jax.lax.rev(sv, (0,))), (0,))
                    incl = cum + sfx_h
                    excl = incl - hv
                    hit = (incl >= k_s) & (excl < k_s)
                    beta_v += jnp.where(hit, j * 16 + lane, 0)
                    cnta_v += jnp.where(hit, excl, 0)
                    suma_v += jnp.where(hit, cums + sfx_s - sv, 0.0)
                    cum = cum + splat_i(jnp.sum(hv))
                    cums = cums + splat_f(jnp.sum(sv))
                beta = splat_i(jnp.sum(beta_v))
                cnt_above = splat_i(jnp.sum(cnta_v))
                sum_above = splat_f(jnp.sum(suma_v))
                return beta, cnt_above, sum_above

            b1, cnta1, suma1 = run_pass(23, k_splat, None, None)
            k2 = k_splat - cnta1
            b2, cnta2, suma2 = run_pass(15, k2, 23, b1)
            kpp = k2 - cnta2
            vhat = plsc.bitcast(
                b1 * (1 << 23) + b2 * (1 << 15) + (1 << 14), jnp.float32
            )
            topk = suma1 + suma2 + kpp.astype(jnp.float32) * vhat
            outv[...] = jnp.where(k_splat > 0, topk, 0.0)
            pltpu.async_copy(outv, out_hbm.at[b], sem).wait()

    return mine(neg_flat, sums_rows)


def _match_half(pred_r, gt_boxes, gt_labels, off, nb):
    return pl.pallas_call(
        _match_kernel,
        grid=(nb,),
        in_specs=[
            pl.BlockSpec(
                (1, _A * (5 + _NC), _ROWS, 128),
                lambda b: (b + off, 0, 0, 0),
            ),
            pl.BlockSpec(memory_space=pltpu.SMEM),
            pl.BlockSpec(memory_space=pltpu.SMEM),
        ],
        out_specs=[
            pl.BlockSpec((1, 1, 128), lambda b: (b, 0, 0)),
            pl.BlockSpec((1, _A * _ROWS, 128), lambda b: (b, 0, 0)),
            pl.BlockSpec((1, _A * _NT * _ROWS, 128), lambda b: (b, 0, 0)),
        ],
        out_shape=[
            jax.ShapeDtypeStruct((nb, 1, 128), jnp.float32),
            jax.ShapeDtypeStruct((nb, _A * _ROWS, 128), jnp.float32),
            jax.ShapeDtypeStruct((nb, _A * _NT * _ROWS, 128), jnp.float32),
        ],
    )(pred_r, gt_boxes, gt_labels)


@jax.jit
def kernel(pred, anchors, gt_boxes, gt_labels):
    del anchors
    pred_r = pred.reshape(_B, _A * (5 + _NC), _ROWS, 128)
    gtl = gt_labels.astype(jnp.int32)
    hb = _B // 2

    sums_a0, neg0, tgt0 = _match_half(
        pred_r, gt_boxes[:hb], gtl[:hb], 0, hb
    )
    sums_a1, neg1, tgt1 = _match_half(
        pred_r, gt_boxes[hb:], gtl[hb:], hb, hb
    )

    topk0 = _sc_mine(neg0.reshape(hb, _NA), sums_a0, 0, hb)
    topk1 = _sc_mine(neg1.reshape(hb, _NA), sums_a1, 1, hb)

    def loss2_half(tgt, off, nb):
        return pl.pallas_call(
            _loss2_kernel,
            grid=(nb,),
            in_specs=[
                pl.BlockSpec(
                    (1, _A * (5 + _NC), _ROWS, 128),
                    lambda b: (b + off, 0, 0, 0),
                ),
                pl.BlockSpec((1, _A * _NT * _ROWS, 128), lambda b: (b, 0, 0)),
            ],
            out_specs=pl.BlockSpec((1, 1, 128), lambda b: (b, 0, 0)),
            out_shape=jax.ShapeDtypeStruct((nb, 1, 128), jnp.float32),
        )(pred_r, tgt)

    sums_b0 = loss2_half(tgt0, 0, hb)
    sums_b1 = loss2_half(tgt1, hb, hb)

    inv_n = 1.0 / float(_B)
    total_loc = (
        jnp.sum(sums_b0[:, 0, 0]) + jnp.sum(sums_b1[:, 0, 0])
    ) * inv_n
    total_obj = (
        jnp.sum(sums_a0[:, 0, 1])
        + jnp.sum(sums_a1[:, 0, 1])
        + jnp.sum(topk0[:, 0])
        + jnp.sum(topk1[:, 0])
    ) * inv_n
    total_cls = (
        jnp.sum(sums_b0[:, 0, 2]) + jnp.sum(sums_b1[:, 0, 2])
    ) * inv_n
    loss = total_loc + total_obj + total_cls
    return loss, total_loc, total_obj, total_cls


# speedup vs baseline: 1.3823x; 1.3823x over previous
import dataclasses
import functools

import jax
import jax.numpy as jnp
from jax.experimental import pallas as pl
from jax.experimental.pallas import tpu as pltpu
from jax.experimental.pallas import tpu_sc as plsc

_B, _H, _W, _A, _NC = 8, 64, 64, 3, 3
_SIZES = (16.0, 32.0, 64.0)
_STRIDE = 8.0
_G = 20
_ROWS = (_H * _W) // 128
_NA = _H * _W * _A
_NT = 6


def _smooth_l1(x, t):
    d = jnp.abs(x - t)
    return jnp.where(d < 1.0, 0.5 * d * d, d - 0.5)


def _anchor_centers():
    col = jax.lax.broadcasted_iota(jnp.int32, (_ROWS, 128), 1)
    row = jax.lax.broadcasted_iota(jnp.int32, (_ROWS, 128), 0)
    x = jnp.remainder(col, _W)
    y = 2 * row + col // _W
    ax = (x.astype(jnp.float32) + 0.5) * _STRIDE
    ay = (y.astype(jnp.float32) + 0.5) * _STRIDE
    return ax, ay, col


def _match_kernel(pred_ref, gtb_ref, gtl_ref, out_ref, neg_ref, tgt_ref):
    b = pl.program_id(0)
    ax, ay, lane = _anchor_centers()
    eps = jnp.float32(1e-6)

    objp_sum = jnp.float32(0.0)
    pos_cnt = jnp.float32(0.0)
    neg_cnt = jnp.float32(0.0)

    for a in range(_A):
        s = _SIZES[a]
        half = s * 0.5
        inv_s = 1.0 / s
        area_a = s * s
        ax1, ay1, ax2, ay2 = ax - half, ay - half, ax + half, ay + half

        best = jnp.full((_ROWS, 128), -1.0, dtype=jnp.float32)
        m_cx = jnp.zeros((_ROWS, 128), dtype=jnp.float32)
        m_cy = jnp.zeros((_ROWS, 128), dtype=jnp.float32)
        m_w = jnp.ones((_ROWS, 128), dtype=jnp.float32)
        m_h = jnp.ones((_ROWS, 128), dtype=jnp.float32)
        m_lab = jnp.zeros((_ROWS, 128), dtype=jnp.float32)

        for g in range(_G):
            gx1 = gtb_ref[b, g, 0]
            gy1 = gtb_ref[b, g, 1]
            gx2 = gtb_ref[b, g, 2]
            gy2 = gtb_ref[b, g, 3]
            glab = gtl_ref[b, g].astype(jnp.float32)
            ix1 = jnp.maximum(ax1, gx1)
            iy1 = jnp.maximum(ay1, gy1)
            ix2 = jnp.minimum(ax2, gx2)
            iy2 = jnp.minimum(ay2, gy2)
            inter = jnp.clip(ix2 - ix1, 0.0) * jnp.clip(iy2 - iy1, 0.0)
            area_g = (gx2 - gx1) * (gy2 - gy1)
            union = area_a + area_g - inter
            iou = inter / jnp.maximum(union, 1e-9)
            upd = iou > best
            best = jnp.where(upd, iou, best)
            m_cx = jnp.where(upd, (gx1 + gx2) * 0.5, m_cx)
            m_cy = jnp.where(upd, (gy1 + gy2) * 0.5, m_cy)
            m_w = jnp.where(upd, jnp.maximum(gx2 - gx1, eps), m_w)
            m_h = jnp.where(upd, jnp.maximum(gy2 - gy1, eps), m_h)
            m_lab = jnp.where(upd, glab, m_lab)

        posf = (best >= 0.5).astype(jnp.float32)
        negm = best < 0.4
        pos_cnt += jnp.sum(posf)
        neg_cnt += jnp.sum(negm.astype(jnp.float32))

        p_obj = pred_ref[0, a * (5 + _NC) + 4, :, :]
        bce = (
            jnp.maximum(p_obj, 0.0)
            - p_obj * posf
            + jnp.log1p(jnp.exp(-jnp.abs(p_obj)))
        )
        objp_sum += jnp.sum(bce * posf)
        neg_ref[0, a * _ROWS : (a + 1) * _ROWS, :] = jnp.where(negm, bce, 0.0)

        t0 = a * _NT * _ROWS
        tgt_ref[0, t0 : t0 + _ROWS, :] = (m_cx - ax) * inv_s
        tgt_ref[0, t0 + _ROWS : t0 + 2 * _ROWS, :] = (m_cy - ay) * inv_s
        tgt_ref[0, t0 + 2 * _ROWS : t0 + 3 * _ROWS, :] = jnp.log(m_w * inv_s)
        tgt_ref[0, t0 + 3 * _ROWS : t0 + 4 * _ROWS, :] = jnp.log(m_h * inv_s)
        tgt_ref[0, t0 + 4 * _ROWS : t0 + 5 * _ROWS, :] = posf
        tgt_ref[0, t0 + 5 * _ROWS : t0 + 6 * _ROWS, :] = m_lab

    k_f = jnp.minimum(
        3.0 * jnp.maximum(1.0, pos_cnt), neg_cnt
    )
    out_row = (
        jnp.where(lane[:1, :] == 1, objp_sum, 0.0)
        + jnp.where(lane[:1, :] == 3, pos_cnt, 0.0)
        + jnp.where(lane[:1, :] == 4, neg_cnt, 0.0)
        + jnp.where(lane[:1, :] == 5, k_f, 0.0)
    )
    out_ref[0, :, :] = out_row


def _loss2_kernel(pred_ref, tgt_ref, out_ref):
    lane = jax.lax.broadcasted_iota(jnp.int32, (1, 128), 1)
    loc_sum = jnp.float32(0.0)
    cls_sum = jnp.float32(0.0)

    for a in range(_A):
        base = a * (5 + _NC)
        t0 = a * _NT * _ROWS
        t_tx = tgt_ref[0, t0 : t0 + _ROWS, :]
        t_ty = tgt_ref[0, t0 + _ROWS : t0 + 2 * _ROWS, :]
        t_tw = tgt_ref[0, t0 + 2 * _ROWS : t0 + 3 * _ROWS, :]
        t_th = tgt_ref[0, t0 + 3 * _ROWS : t0 + 4 * _ROWS, :]
        posf = tgt_ref[0, t0 + 4 * _ROWS : t0 + 5 * _ROWS, :]
        m_lab = tgt_ref[0, t0 + 5 * _ROWS : t0 + 6 * _ROWS, :]

        loc_plane = (
            _smooth_l1(pred_ref[0, base + 0, :, :], t_tx)
            + _smooth_l1(pred_ref[0, base + 1, :, :], t_ty)
            + _smooth_l1(pred_ref[0, base + 2, :, :], t_tw)
            + _smooth_l1(pred_ref[0, base + 3, :, :], t_th)
        )
        loc_sum += jnp.sum(loc_plane * posf)

        c0 = pred_ref[0, base + 5, :, :]
        c1 = pred_ref[0, base + 6, :, :]
        c2 = pred_ref[0, base + 7, :, :]
        m = jnp.maximum(jnp.maximum(c0, c1), c2)
        lse = m + jnp.log(
            jnp.exp(c0 - m) + jnp.exp(c1 - m) + jnp.exp(c2 - m)
        )
        picked = jnp.where(m_lab < 0.5, c0, jnp.where(m_lab < 1.5, c1, c2))
        cls_sum += jnp.sum((lse - picked) * posf)

    out_row = jnp.where(lane == 0, loc_sum, 0.0) + jnp.where(
        lane == 2, cls_sum, 0.0
    )
    out_ref[0, :, :] = out_row


_NSLOT = 2


def _sc_mine(neg_flat, sums_rows):
    mesh = plsc.VectorSubcoreMesh(core_axis_name="c", subcore_axis_name="s")
    cp = pltpu.CompilerParams()
    if "needs_layout_passes" in pltpu.CompilerParams.__dataclass_fields__:
        cp = dataclasses.replace(cp, needs_layout_passes=False)
    hsize = _NSLOT * 16 * 256

    @functools.partial(
        pl.kernel,
        mesh=mesh,
        compiler_params=cp,
        out_type=jax.ShapeDtypeStruct((_B, 16), jnp.float32),
        scratch_types=[
            pltpu.VMEM((_NA,), jnp.float32),
            pltpu.VMEM((hsize,), jnp.int32),
            pltpu.VMEM((hsize,), jnp.float32),
            pltpu.VMEM((1, 128), jnp.float32),
            pltpu.VMEM((16,), jnp.float32),
            pltpu.SemaphoreType.DMA,
        ],
    )
    def mine(neg_hbm, row_hbm, out_hbm, data, hist, sums, krow, outv, sem):
        wid = jax.lax.axis_index("s") * 2 + jax.lax.axis_index("c")

        @pl.when(wid < _B)
        def _():
            b = wid
            pltpu.async_copy(neg_hbm.at[b], data, sem).wait()
            pltpu.async_copy(row_hbm.at[b], krow, sem).wait()
            lane = jax.lax.iota(jnp.int32, 16)
            lane256 = lane * 256
            lane256_hi = lane256 + 4096
            ones = jnp.full((16,), 1, jnp.int32)
            zero16i = jnp.zeros((16,), jnp.int32)
            zero16f = jnp.zeros((16,), jnp.float32)
            k_splat = plsc.load_gather(
                krow, [zero16i, jnp.full((16,), 5, jnp.int32)]
            ).astype(jnp.int32)

            def splat_i(v):
                return jnp.full((16,), v, jnp.int32)

            def splat_f(v):
                return jnp.full((16,), v, jnp.float32)

            def run_pass(shift, k_s, prev_shift, prev_beta):
                @plsc.parallel_loop(0, hsize, step=16)
                def _(i):
                    hist[pl.ds(i, 16)] = zero16i
                    sums[pl.ds(i, 16)] = zero16f

                @plsc.parallel_loop(0, _NA, step=32)
                def _(i):
                    for base_v in (lane256, lane256_hi):
                        v = data[pl.ds(i, 16)]
                        i = i + 16
                        bits = plsc.bitcast(v, jnp.int32)
                        if shift == 23:
                            d = jax.lax.shift_right_logical(bits, 23)
                        else:
                            d = jnp.bitwise_and(
                                jax.lax.shift_right_logical(bits, shift), 255
                            )
                        idx = base_v + d
                        if prev_shift is None:
                            plsc.addupdate_scatter(hist, [idx], ones)
                            plsc.addupdate_scatter(sums, [idx], v)
                        else:
                            pd = jax.lax.shift_right_logical(bits, prev_shift)
                            m = pd == prev_beta
                            plsc.addupdate_scatter(hist, [idx], ones, mask=m)
                            plsc.addupdate_scatter(sums, [idx], v, mask=m)

                cum = splat_i(0)
                cums = splat_f(0.0)
                beta_v = splat_i(0)
                cnta_v = splat_i(0)
                suma_v = splat_f(0.0)
                for j in reversed(range(16)):
                    hv = hist[pl.ds(j * 16, 16)]
                    sv = sums[pl.ds(j * 16, 16)]
                    for t in range(1, 16 * _NSLOT):
                        hv = hv + hist[pl.ds(t * 256 + j * 16, 16)]
                        sv = sv + sums[pl.ds(t * 256 + j * 16, 16)]
                    sfx_h = jax.lax.rev(plsc.cumsum(jax.lax.rev(hv, (0,))), (0,))
                    sfx_s = jax.lax.rev(plsc.cumsum(jax.lax.rev(sv, (0,))), (0,))
                    incl = cum + sfx_h
                    excl = incl - hv
                    hit = (incl >= k_s) & (excl < k_s)
                    beta_v += jnp.where(hit, j * 16 + lane, 0)
                    cnta_v += jnp.where(hit, excl, 0)
                    suma_v += jnp.where(hit, cums + sfx_s - sv, 0.0)
                    cum = cum + splat_i(jnp.sum(hv))
                    cums = cums + splat_f(jnp.sum(sv))
                beta = splat_i(jnp.sum(beta_v))
                cnt_above = splat_i(jnp.sum(cnta_v))
                sum_above = splat_f(jnp.sum(suma_v))
                return beta, cnt_above, sum_above

            b1, cnta1, suma1 = run_pass(23, k_splat, None, None)
            k2 = k_splat - cnta1
            b2, cnta2, suma2 = run_pass(15, k2, 23, b1)
            kpp = k2 - cnta2
            vhat = plsc.bitcast(
                b1 * (1 << 23) + b2 * (1 << 15) + (1 << 14), jnp.float32
            )
            topk = suma1 + suma2 + kpp.astype(jnp.float32) * vhat
            outv[...] = jnp.where(k_splat > 0, topk, 0.0)
            pltpu.async_copy(outv, out_hbm.at[b], sem).wait()

    return mine(neg_flat, sums_rows)


@jax.jit
def kernel(pred, anchors, gt_boxes, gt_labels):
    del anchors
    pred_r = pred.reshape(_B, _A * (5 + _NC), _ROWS, 128)
    sums_a, neg, tgt = pl.pallas_call(
        _match_kernel,
        grid=(_B,),
        in_specs=[
            pl.BlockSpec(
                (1, _A * (5 + _NC), _ROWS, 128), lambda b: (b, 0, 0, 0)
            ),
            pl.BlockSpec(memory_space=pltpu.SMEM),
            pl.BlockSpec(memory_space=pltpu.SMEM),
        ],
        out_specs=[
            pl.BlockSpec((1, 1, 128), lambda b: (b, 0, 0)),
            pl.BlockSpec((1, _A * _ROWS, 128), lambda b: (b, 0, 0)),
            pl.BlockSpec((1, _A * _NT * _ROWS, 128), lambda b: (b, 0, 0)),
        ],
        out_shape=[
            jax.ShapeDtypeStruct((_B, 1, 128), jnp.float32),
            jax.ShapeDtypeStruct((_B, _A * _ROWS, 128), jnp.float32),
            jax.ShapeDtypeStruct((_B, _A * _NT * _ROWS, 128), jnp.float32),
        ],
    )(pred_r, gt_boxes, gt_labels.astype(jnp.int32))

    topk_rows = _sc_mine(neg.reshape(_B, _NA), sums_a)

    sums_b = pl.pallas_call(
        _loss2_kernel,
        grid=(_B,),
        in_specs=[
            pl.BlockSpec(
                (1, _A * (5 + _NC), _ROWS, 128), lambda b: (b, 0, 0, 0)
            ),
            pl.BlockSpec((1, _A * _NT * _ROWS, 128), lambda b: (b, 0, 0)),
        ],
        out_specs=pl.BlockSpec((1, 1, 128), lambda b: (b, 0, 0)),
        out_shape=jax.ShapeDtypeStruct((_B, 1, 128), jnp.float32),
    )(pred_r, tgt)

    topk = topk_rows[:, 0]
    inv_n = 1.0 / float(_B)
    total_loc = jnp.sum(sums_b[:, 0, 0]) * inv_n
    total_obj = (jnp.sum(sums_a[:, 0, 1]) + jnp.sum(topk)) * inv_n
    total_cls = jnp.sum(sums_b[:, 0, 2]) * inv_n
    loss = total_loc + total_obj + total_cls
    return loss, total_loc, total_obj, total_cls


# speedup vs baseline: 1.5057x; 1.0893x over previous
import dataclasses
import functools

import jax
import jax.numpy as jnp
from jax.experimental import pallas as pl
from jax.experimental.pallas import tpu as pltpu
from jax.experimental.pallas import tpu_sc as plsc

_B, _H, _W, _A, _NC = 8, 64, 64, 3, 3
_SIZES = (16.0, 32.0, 64.0)
_STRIDE = 8.0
_G = 20
_ROWS = (_H * _W) // 128
_NA = _H * _W * _A
_NT = 6


def _smooth_l1(x, t):
    d = jnp.abs(x - t)
    return jnp.where(d < 1.0, 0.5 * d * d, d - 0.5)


def _anchor_centers():
    col = jax.lax.broadcasted_iota(jnp.int32, (_ROWS, 128), 1)
    row = jax.lax.broadcasted_iota(jnp.int32, (_ROWS, 128), 0)
    x = jnp.remainder(col, _W)
    y = 2 * row + col // _W
    ax = (x.astype(jnp.float32) + 0.5) * _STRIDE
    ay = (y.astype(jnp.float32) + 0.5) * _STRIDE
    return ax, ay, col


def _match_kernel(pred_ref, gtb_ref, gtl_ref, out_ref, neg_ref, tgt_ref):
    b = pl.program_id(0)
    ax, ay, lane = _anchor_centers()
    eps = jnp.float32(1e-6)

    objp_sum = jnp.float32(0.0)
    pos_cnt = jnp.float32(0.0)
    neg_cnt = jnp.float32(0.0)

    for a in range(_A):
        s = _SIZES[a]
        half = s * 0.5
        inv_s = 1.0 / s
        area_a = s * s
        ax1, ay1, ax2, ay2 = ax - half, ay - half, ax + half, ay + half

        best = jnp.full((_ROWS, 128), -1.0, dtype=jnp.float32)
        m_cx = jnp.zeros((_ROWS, 128), dtype=jnp.float32)
        m_cy = jnp.zeros((_ROWS, 128), dtype=jnp.float32)
        m_w = jnp.ones((_ROWS, 128), dtype=jnp.float32)
        m_h = jnp.ones((_ROWS, 128), dtype=jnp.float32)
        m_lab = jnp.zeros((_ROWS, 128), dtype=jnp.float32)

        for g in range(_G):
            gx1 = gtb_ref[b, g, 0]
            gy1 = gtb_ref[b, g, 1]
            gx2 = gtb_ref[b, g, 2]
            gy2 = gtb_ref[b, g, 3]
            glab = gtl_ref[b, g].astype(jnp.float32)
            ix1 = jnp.maximum(ax1, gx1)
            iy1 = jnp.maximum(ay1, gy1)
            ix2 = jnp.minimum(ax2, gx2)
            iy2 = jnp.minimum(ay2, gy2)
            inter = jnp.clip(ix2 - ix1, 0.0) * jnp.clip(iy2 - iy1, 0.0)
            area_g = (gx2 - gx1) * (gy2 - gy1)
            union = area_a + area_g - inter
            iou = inter / jnp.maximum(union, 1e-9)
            upd = iou > best
            best = jnp.where(upd, iou, best)
            m_cx = jnp.where(upd, (gx1 + gx2) * 0.5, m_cx)
            m_cy = jnp.where(upd, (gy1 + gy2) * 0.5, m_cy)
            m_w = jnp.where(upd, jnp.maximum(gx2 - gx1, eps), m_w)
            m_h = jnp.where(upd, jnp.maximum(gy2 - gy1, eps), m_h)
            m_lab = jnp.where(upd, glab, m_lab)

        posf = (best >= 0.5).astype(jnp.float32)
        negm = best < 0.4
        pos_cnt += jnp.sum(posf)
        neg_cnt += jnp.sum(negm.astype(jnp.float32))

        p_obj = pred_ref[0, a * (5 + _NC) + 4, :, :]
        bce = (
            jnp.maximum(p_obj, 0.0)
            - p_obj * posf
            + jnp.log1p(jnp.exp(-jnp.abs(p_obj)))
        )
        objp_sum += jnp.sum(bce * posf)
        neg_ref[0, a * _ROWS : (a + 1) * _ROWS, :] = jnp.where(negm, bce, 0.0)

        t0 = a * _NT * _ROWS
        tgt_ref[0, t0 : t0 + _ROWS, :] = (m_cx - ax) * inv_s
        tgt_ref[0, t0 + _ROWS : t0 + 2 * _ROWS, :] = (m_cy - ay) * inv_s
        tgt_ref[0, t0 + 2 * _ROWS : t0 + 3 * _ROWS, :] = jnp.log(m_w * inv_s)
        tgt_ref[0, t0 + 3 * _ROWS : t0 + 4 * _ROWS, :] = jnp.log(m_h * inv_s)
        tgt_ref[0, t0 + 4 * _ROWS : t0 + 5 * _ROWS, :] = posf
        tgt_ref[0, t0 + 5 * _ROWS : t0 + 6 * _ROWS, :] = m_lab

    k_f = jnp.minimum(
        3.0 * jnp.maximum(1.0, pos_cnt), neg_cnt
    )
    out_row = (
        jnp.where(lane[:1, :] == 1, objp_sum, 0.0)
        + jnp.where(lane[:1, :] == 3, pos_cnt, 0.0)
        + jnp.where(lane[:1, :] == 4, neg_cnt, 0.0)
        + jnp.where(lane[:1, :] == 5, k_f, 0.0)
    )
    out_ref[0, :, :] = out_row


def _loss2_kernel(pred_ref, tgt_ref, out_ref):
    lane = jax.lax.broadcasted_iota(jnp.int32, (1, 128), 1)
    loc_sum = jnp.float32(0.0)
    cls_sum = jnp.float32(0.0)

    for a in range(_A):
        base = a * (5 + _NC)
        t0 = a * _NT * _ROWS
        t_tx = tgt_ref[0, t0 : t0 + _ROWS, :]
        t_ty = tgt_ref[0, t0 + _ROWS : t0 + 2 * _ROWS, :]
        t_tw = tgt_ref[0, t0 + 2 * _ROWS : t0 + 3 * _ROWS, :]
        t_th = tgt_ref[0, t0 + 3 * _ROWS : t0 + 4 * _ROWS, :]
        posf = tgt_ref[0, t0 + 4 * _ROWS : t0 + 5 * _ROWS, :]
        m_lab = tgt_ref[0, t0 + 5 * _ROWS : t0 + 6 * _ROWS, :]

        loc_plane = (
            _smooth_l1(pred_ref[0, base + 0, :, :], t_tx)
            + _smooth_l1(pred_ref[0, base + 1, :, :], t_ty)
            + _smooth_l1(pred_ref[0, base + 2, :, :], t_tw)
            + _smooth_l1(pred_ref[0, base + 3, :, :], t_th)
        )
        loc_sum += jnp.sum(loc_plane * posf)

        c0 = pred_ref[0, base + 5, :, :]
        c1 = pred_ref[0, base + 6, :, :]
        c2 = pred_ref[0, base + 7, :, :]
        m = jnp.maximum(jnp.maximum(c0, c1), c2)
        lse = m + jnp.log(
            jnp.exp(c0 - m) + jnp.exp(c1 - m) + jnp.exp(c2 - m)
        )
        picked = jnp.where(m_lab < 0.5, c0, jnp.where(m_lab < 1.5, c1, c2))
        cls_sum += jnp.sum((lse - picked) * posf)

    out_row = jnp.where(lane == 0, loc_sum, 0.0) + jnp.where(
        lane == 2, cls_sum, 0.0
    )
    out_ref[0, :, :] = out_row


_NSLOT = 2


def _sc_mine(neg_flat, sums_rows):
    mesh = plsc.VectorSubcoreMesh(core_axis_name="c", subcore_axis_name="s")
    cp = pltpu.CompilerParams()
    if "needs_layout_passes" in pltpu.CompilerParams.__dataclass_fields__:
        cp = dataclasses.replace(cp, needs_layout_passes=False)
    hsize = _NSLOT * 16 * 256

    @functools.partial(
        pl.kernel,
        mesh=mesh,
        compiler_params=cp,
        out_type=jax.ShapeDtypeStruct((_B, 16), jnp.float32),
        scratch_types=[
            pltpu.VMEM((_NA,), jnp.float32),
            pltpu.VMEM((hsize,), jnp.int32),
            pltpu.VMEM((1, 128), jnp.float32),
            pltpu.VMEM((16,), jnp.float32),
            pltpu.SemaphoreType.DMA,
        ],
    )
    def mine(neg_hbm, row_hbm, out_hbm, data, hist, krow, outv, sem):
        wid = jax.lax.axis_index("s") * 2 + jax.lax.axis_index("c")

        @pl.when(wid < _B)
        def _():
            b = wid
            pltpu.async_copy(neg_hbm.at[b], data, sem).wait()
            pltpu.async_copy(row_hbm.at[b], krow, sem).wait()
            lane = jax.lax.iota(jnp.int32, 16)
            lane256 = lane * 256
            lane256_hi = lane256 + 4096
            ones = jnp.full((16,), 1, jnp.int32)
            zero16i = jnp.zeros((16,), jnp.int32)
            zero16f = jnp.zeros((16,), jnp.float32)
            k_splat = plsc.load_gather(
                krow, [zero16i, jnp.full((16,), 5, jnp.int32)]
            ).astype(jnp.int32)

            def splat_i(v):
                return jnp.full((16,), v, jnp.int32)

            def splat_f(v):
                return jnp.full((16,), v, jnp.float32)

            def run_pass(shift, k_s, prev_shift, prev_beta):
                @plsc.parallel_loop(0, hsize, step=16)
                def _(i):
                    hist[pl.ds(i, 16)] = zero16i

                @plsc.parallel_loop(0, _NA, step=32)
                def _(i):
                    for base_v in (lane256, lane256_hi):
                        v = data[pl.ds(i, 16)]
                        i = i + 16
                        bits = plsc.bitcast(v, jnp.int32)
                        if shift == 23:
                            d = jax.lax.shift_right_logical(bits, 23)
                        else:
                            d = jnp.bitwise_and(
                                jax.lax.shift_right_logical(bits, shift), 255
                            )
                        idx = base_v + d
                        if prev_shift is None:
                            plsc.addupdate_scatter(hist, [idx], ones)
                        else:
                            pd = jax.lax.shift_right_logical(bits, prev_shift)
                            m = pd == prev_beta
                            plsc.addupdate_scatter(hist, [idx], ones, mask=m)

                cum = splat_i(0)
                beta_v = splat_i(0)
                cnta_v = splat_i(0)
                for j in reversed(range(16)):
                    hv = hist[pl.ds(j * 16, 16)]
                    for t in range(1, 16 * _NSLOT):
                        hv = hv + hist[pl.ds(t * 256 + j * 16, 16)]
                    sfx_h = jax.lax.rev(plsc.cumsum(jax.lax.rev(hv, (0,))), (0,))
                    incl = cum + sfx_h
                    excl = incl - hv
                    hit = (incl >= k_s) & (excl < k_s)
                    beta_v += jnp.where(hit, j * 16 + lane, 0)
                    cnta_v += jnp.where(hit, excl, 0)
                    cum = cum + splat_i(jnp.sum(hv))
                beta = splat_i(jnp.sum(beta_v))
                cnt_above = splat_i(jnp.sum(cnta_v))
                return beta, cnt_above

            b1, cnta1 = run_pass(23, k_splat, None, None)
            k2 = k_splat - cnta1
            b2, _ = run_pass(15, k2, 23, b1)

            hi_bits = b1 * (1 << 23) + (b2 + 1) * (1 << 15)

            @plsc.parallel_loop(
                0, _NA, step=16, carry=(zero16f, zero16i)
            )
            def acc(i, carry):
                sv, cv = carry
                v = data[pl.ds(i, 16)]
                bits = plsc.bitcast(v, jnp.int32)
                ge = bits >= hi_bits
                return (
                    sv + jnp.where(ge, v, 0.0),
                    cv + jnp.where(ge, 1, 0),
                )

            sum_above = splat_f(jnp.sum(acc[0]))
            cnt_above = splat_i(jnp.sum(acc[1]))
            kpp = k_splat - cnt_above
            vhat = plsc.bitcast(
                b1 * (1 << 23) + b2 * (1 << 15) + (1 << 14), jnp.float32
            )
            topk = sum_above + kpp.astype(jnp.float32) * vhat
            outv[...] = jnp.where(k_splat > 0, topk, 0.0)
            pltpu.async_copy(outv, out_hbm.at[b], sem).wait()

    return mine(neg_flat, sums_rows)


@jax.jit
def kernel(pred, anchors, gt_boxes, gt_labels):
    del anchors
    pred_r = pred.reshape(_B, _A * (5 + _NC), _ROWS, 128)
    sums_a, neg, tgt = pl.pallas_call(
        _match_kernel,
        grid=(_B,),
        in_specs=[
            pl.BlockSpec(
                (1, _A * (5 + _NC), _ROWS, 128), lambda b: (b, 0, 0, 0)
            ),
            pl.BlockSpec(memory_space=pltpu.SMEM),
            pl.BlockSpec(memory_space=pltpu.SMEM),
        ],
        out_specs=[
            pl.BlockSpec((1, 1, 128), lambda b: (b, 0, 0)),
            pl.BlockSpec((1, _A * _ROWS, 128), lambda b: (b, 0, 0)),
            pl.BlockSpec((1, _A * _NT * _ROWS, 128), lambda b: (b, 0, 0)),
        ],
        out_shape=[
            jax.ShapeDtypeStruct((_B, 1, 128), jnp.float32),
            jax.ShapeDtypeStruct((_B, _A * _ROWS, 128), jnp.float32),
            jax.ShapeDtypeStruct((_B, _A * _NT * _ROWS, 128), jnp.float32),
        ],
    )(pred_r, gt_boxes, gt_labels.astype(jnp.int32))

    topk_rows = _sc_mine(neg.reshape(_B, _NA), sums_a)

    sums_b = pl.pallas_call(
        _loss2_kernel,
        grid=(_B,),
        in_specs=[
            pl.BlockSpec(
                (1, _A * (5 + _NC), _ROWS, 128), lambda b: (b, 0, 0, 0)
            ),
            pl.BlockSpec((1, _A * _NT * _ROWS, 128), lambda b: (b, 0, 0)),
        ],
        out_specs=pl.BlockSpec((1, 1, 128), lambda b: (b, 0, 0)),
        out_shape=jax.ShapeDtypeStruct((_B, 1, 128), jnp.float32),
    )(pred_r, tgt)

    topk = topk_rows[:, 0]
    inv_n = 1.0 / float(_B)
    total_loc = jnp.sum(sums_b[:, 0, 0]) * inv_n
    total_obj = (jnp.sum(sums_a[:, 0, 1]) + jnp.sum(topk)) * inv_n
    total_cls = jnp.sum(sums_b[:, 0, 2]) * inv_n
    loss = total_loc + total_obj + total_cls
    return loss, total_loc, total_obj, total_cls


# speedup vs baseline: 1.5063x; 1.0004x over previous
import dataclasses
import functools

import jax
import jax.numpy as jnp
from jax.experimental import pallas as pl
from jax.experimental.pallas import tpu as pltpu
from jax.experimental.pallas import tpu_sc as plsc

_B, _H, _W, _A, _NC = 8, 64, 64, 3, 3
_SIZES = (16.0, 32.0, 64.0)
_STRIDE = 8.0
_G = 20
_ROWS = (_H * _W) // 128
_NA = _H * _W * _A
_NT = 6


def _smooth_l1(x, t):
    d = jnp.abs(x - t)
    return jnp.where(d < 1.0, 0.5 * d * d, d - 0.5)


def _anchor_centers():
    col = jax.lax.broadcasted_iota(jnp.int32, (_ROWS, 128), 1)
    row = jax.lax.broadcasted_iota(jnp.int32, (_ROWS, 128), 0)
    x = jnp.remainder(col, _W)
    y = 2 * row + col // _W
    ax = (x.astype(jnp.float32) + 0.5) * _STRIDE
    ay = (y.astype(jnp.float32) + 0.5) * _STRIDE
    return ax, ay, col


def _match_kernel(pred_ref, gtb_ref, gtl_ref, out_ref, neg_ref):
    b = pl.program_id(0)
    ax, ay, lane = _anchor_centers()
    eps = jnp.float32(1e-6)

    objp_sum = jnp.float32(0.0)
    pos_cnt = jnp.float32(0.0)
    neg_cnt = jnp.float32(0.0)
    loc_sum = jnp.float32(0.0)
    cls_sum = jnp.float32(0.0)

    for a in range(_A):
        s = _SIZES[a]
        half = s * 0.5
        inv_s = 1.0 / s
        area_a = s * s
        ax1, ay1, ax2, ay2 = ax - half, ay - half, ax + half, ay + half

        best = jnp.full((_ROWS, 128), -1.0, dtype=jnp.float32)
        m_cx = jnp.zeros((_ROWS, 128), dtype=jnp.float32)
        m_cy = jnp.zeros((_ROWS, 128), dtype=jnp.float32)
        m_w = jnp.ones((_ROWS, 128), dtype=jnp.float32)
        m_h = jnp.ones((_ROWS, 128), dtype=jnp.float32)
        m_lab = jnp.zeros((_ROWS, 128), dtype=jnp.float32)

        for g in range(_G):
            gx1 = gtb_ref[b, g, 0]
            gy1 = gtb_ref[b, g, 1]
            gx2 = gtb_ref[b, g, 2]
            gy2 = gtb_ref[b, g, 3]
            glab = gtl_ref[b, g].astype(jnp.float32)
            ix1 = jnp.maximum(ax1, gx1)
            iy1 = jnp.maximum(ay1, gy1)
            ix2 = jnp.minimum(ax2, gx2)
            iy2 = jnp.minimum(ay2, gy2)
            inter = jnp.clip(ix2 - ix1, 0.0) * jnp.clip(iy2 - iy1, 0.0)
            area_g = (gx2 - gx1) * (gy2 - gy1)
            union = area_a + area_g - inter
            iou = inter / jnp.maximum(union, 1e-9)
            upd = iou > best
            best = jnp.where(upd, iou, best)
            m_cx = jnp.where(upd, (gx1 + gx2) * 0.5, m_cx)
            m_cy = jnp.where(upd, (gy1 + gy2) * 0.5, m_cy)
            m_w = jnp.where(upd, jnp.maximum(gx2 - gx1, eps), m_w)
            m_h = jnp.where(upd, jnp.maximum(gy2 - gy1, eps), m_h)
            m_lab = jnp.where(upd, glab, m_lab)

        posf = (best >= 0.5).astype(jnp.float32)
        negm = best < 0.4
        pos_cnt += jnp.sum(posf)
        neg_cnt += jnp.sum(negm.astype(jnp.float32))

        p_obj = pred_ref[0, a * (5 + _NC) + 4, :, :]
        bce = (
            jnp.maximum(p_obj, 0.0)
            - p_obj * posf
            + jnp.log1p(jnp.exp(-jnp.abs(p_obj)))
        )
        objp_sum += jnp.sum(bce * posf)
        neg_ref[0, a * _ROWS : (a + 1) * _ROWS, :] = jnp.where(negm, bce, 0.0)

        base = a * (5 + _NC)
        loc_plane = (
            _smooth_l1(pred_ref[0, base + 0, :, :], (m_cx - ax) * inv_s)
            + _smooth_l1(pred_ref[0, base + 1, :, :], (m_cy - ay) * inv_s)
            + _smooth_l1(pred_ref[0, base + 2, :, :], jnp.log(m_w * inv_s))
            + _smooth_l1(pred_ref[0, base + 3, :, :], jnp.log(m_h * inv_s))
        )
        loc_sum += jnp.sum(loc_plane * posf)

        c0 = pred_ref[0, base + 5, :, :]
        c1 = pred_ref[0, base + 6, :, :]
        c2 = pred_ref[0, base + 7, :, :]
        m = jnp.maximum(jnp.maximum(c0, c1), c2)
        lse = m + jnp.log(
            jnp.exp(c0 - m) + jnp.exp(c1 - m) + jnp.exp(c2 - m)
        )
        picked = jnp.where(m_lab < 0.5, c0, jnp.where(m_lab < 1.5, c1, c2))
        cls_sum += jnp.sum((lse - picked) * posf)

    k_f = jnp.minimum(
        3.0 * jnp.maximum(1.0, pos_cnt), neg_cnt
    )
    out_row = (
        jnp.where(lane[:1, :] == 0, loc_sum, 0.0)
        + jnp.where(lane[:1, :] == 1, objp_sum, 0.0)
        + jnp.where(lane[:1, :] == 2, cls_sum, 0.0)
        + jnp.where(lane[:1, :] == 3, pos_cnt, 0.0)
        + jnp.where(lane[:1, :] == 4, neg_cnt, 0.0)
        + jnp.where(lane[:1, :] == 5, k_f, 0.0)
    )
    out_ref[0, :, :] = out_row


def _loss2_kernel(pred_ref, tgt_ref, out_ref):
    lane = jax.lax.broadcasted_iota(jnp.int32, (1, 128), 1)
    loc_sum = jnp.float32(0.0)
    cls_sum = jnp.float32(0.0)

    for a in range(_A):
        base = a * (5 + _NC)
        t0 = a * _NT * _ROWS
        t_tx = tgt_ref[0, t0 : t0 + _ROWS, :]
        t_ty = tgt_ref[0, t0 + _ROWS : t0 + 2 * _ROWS, :]
        t_tw = tgt_ref[0, t0 + 2 * _ROWS : t0 + 3 * _ROWS, :]
        t_th = tgt_ref[0, t0 + 3 * _ROWS : t0 + 4 * _ROWS, :]
        posf = tgt_ref[0, t0 + 4 * _ROWS : t0 + 5 * _ROWS, :]
        m_lab = tgt_ref[0, t0 + 5 * _ROWS : t0 + 6 * _ROWS, :]

        loc_plane = (
            _smooth_l1(pred_ref[0, base + 0, :, :], t_tx)
            + _smooth_l1(pred_ref[0, base + 1, :, :], t_ty)
            + _smooth_l1(pred_ref[0, base + 2, :, :], t_tw)
            + _smooth_l1(pred_ref[0, base + 3, :, :], t_th)
        )
        loc_sum += jnp.sum(loc_plane * posf)

        c0 = pred_ref[0, base + 5, :, :]
        c1 = pred_ref[0, base + 6, :, :]
        c2 = pred_ref[0, base + 7, :, :]
        m = jnp.maximum(jnp.maximum(c0, c1), c2)
        lse = m + jnp.log(
            jnp.exp(c0 - m) + jnp.exp(c1 - m) + jnp.exp(c2 - m)
        )
        picked = jnp.where(m_lab < 0.5, c0, jnp.where(m_lab < 1.5, c1, c2))
        cls_sum += jnp.sum((lse - picked) * posf)

    out_row = jnp.where(lane == 0, loc_sum, 0.0) + jnp.where(
        lane == 2, cls_sum, 0.0
    )
    out_ref[0, :, :] = out_row


_NSLOT = 2


def _sc_mine(neg_flat, sums_rows):
    mesh = plsc.VectorSubcoreMesh(core_axis_name="c", subcore_axis_name="s")
    cp = pltpu.CompilerParams()
    if "needs_layout_passes" in pltpu.CompilerParams.__dataclass_fields__:
        cp = dataclasses.replace(cp, needs_layout_passes=False)
    hsize = _NSLOT * 16 * 256

    @functools.partial(
        pl.kernel,
        mesh=mesh,
        compiler_params=cp,
        out_type=jax.ShapeDtypeStruct((_B, 16), jnp.float32),
        scratch_types=[
            pltpu.VMEM((_NA,), jnp.float32),
            pltpu.VMEM((hsize,), jnp.int32),
            pltpu.VMEM((1, 128), jnp.float32),
            pltpu.VMEM((16,), jnp.float32),
            pltpu.SemaphoreType.DMA,
        ],
    )
    def mine(neg_hbm, row_hbm, out_hbm, data, hist, krow, outv, sem):
        wid = jax.lax.axis_index("s") * 2 + jax.lax.axis_index("c")

        @pl.when(wid < _B)
        def _():
            b = wid
            pltpu.async_copy(neg_hbm.at[b], data, sem).wait()
            pltpu.async_copy(row_hbm.at[b], krow, sem).wait()
            lane = jax.lax.iota(jnp.int32, 16)
            lane256 = lane * 256
            lane256_hi = lane256 + 4096
            ones = jnp.full((16,), 1, jnp.int32)
            zero16i = jnp.zeros((16,), jnp.int32)
            zero16f = jnp.zeros((16,), jnp.float32)
            k_splat = plsc.load_gather(
                krow, [zero16i, jnp.full((16,), 5, jnp.int32)]
            ).astype(jnp.int32)

            def splat_i(v):
                return jnp.full((16,), v, jnp.int32)

            def splat_f(v):
                return jnp.full((16,), v, jnp.float32)

            def run_pass(shift, k_s, prev_shift, prev_beta):
                @plsc.parallel_loop(0, hsize, step=16)
                def _(i):
                    hist[pl.ds(i, 16)] = zero16i

                @plsc.parallel_loop(0, _NA, step=32)
                def _(i):
                    for base_v in (lane256, lane256_hi):
                        v = data[pl.ds(i, 16)]
                        i = i + 16
                        bits = plsc.bitcast(v, jnp.int32)
                        if shift == 23:
                            d = jax.lax.shift_right_logical(bits, 23)
                        else:
                            d = jnp.bitwise_and(
                                jax.lax.shift_right_logical(bits, shift), 255
                            )
                        idx = base_v + d
                        if prev_shift is None:
                            plsc.addupdate_scatter(hist, [idx], ones)
                        else:
                            pd = jax.lax.shift_right_logical(bits, prev_shift)
                            m = pd == prev_beta
                            plsc.addupdate_scatter(hist, [idx], ones, mask=m)

                cum = splat_i(0)
                beta_v = splat_i(0)
                cnta_v = splat_i(0)
                for j in reversed(range(16)):
                    hv = hist[pl.ds(j * 16, 16)]
                    for t in range(1, 16 * _NSLOT):
                        hv = hv + hist[pl.ds(t * 256 + j * 16, 16)]
                    sfx_h = jax.lax.rev(plsc.cumsum(jax.lax.rev(hv, (0,))), (0,))
                    incl = cum + sfx_h
                    excl = incl - hv
                    hit = (incl >= k_s) & (excl < k_s)
                    beta_v += jnp.where(hit, j * 16 + lane, 0)
                    cnta_v += jnp.where(hit, excl, 0)
                    cum = cum + splat_i(jnp.sum(hv))
                beta = splat_i(jnp.sum(beta_v))
                cnt_above = splat_i(jnp.sum(cnta_v))
                return beta, cnt_above

            b1, cnta1 = run_pass(23, k_splat, None, None)
            k2 = k_splat - cnta1
            b2, _ = run_pass(15, k2, 23, b1)

            hi_bits = b1 * (1 << 23) + (b2 + 1) * (1 << 15)

            @plsc.parallel_loop(
                0, _NA, step=16, carry=(zero16f, zero16i)
            )
            def acc(i, carry):
                sv, cv = carry
                v = data[pl.ds(i, 16)]
                bits = plsc.bitcast(v, jnp.int32)
                ge = bits >= hi_bits
                return (
                    sv + jnp.where(ge, v, 0.0),
                    cv + jnp.where(ge, 1, 0),
                )

            sum_above = splat_f(jnp.sum(acc[0]))
            cnt_above = splat_i(jnp.sum(acc[1]))
            kpp = k_splat - cnt_above
            vhat = plsc.bitcast(
                b1 * (1 << 23) + b2 * (1 << 15) + (1 << 14), jnp.float32
            )
            topk = sum_above + kpp.astype(jnp.float32) * vhat
            outv[...] = jnp.where(k_splat > 0, topk, 0.0)
            pltpu.async_copy(outv, out_hbm.at[b], sem).wait()

    return mine(neg_flat, sums_rows)


@jax.jit
def kernel(pred, anchors, gt_boxes, gt_labels):
    del anchors
    pred_r = pred.reshape(_B, _A * (5 + _NC), _ROWS, 128)
    sums_a, neg = pl.pallas_call(
        _match_kernel,
        grid=(_B,),
        in_specs=[
            pl.BlockSpec(
                (1, _A * (5 + _NC), _ROWS, 128), lambda b: (b, 0, 0, 0)
            ),
            pl.BlockSpec(memory_space=pltpu.SMEM),
            pl.BlockSpec(memory_space=pltpu.SMEM),
        ],
        out_specs=[
            pl.BlockSpec((1, 1, 128), lambda b: (b, 0, 0)),
            pl.BlockSpec((1, _A * _ROWS, 128), lambda b: (b, 0, 0)),
        ],
        out_shape=[
            jax.ShapeDtypeStruct((_B, 1, 128), jnp.float32),
            jax.ShapeDtypeStruct((_B, _A * _ROWS, 128), jnp.float32),
        ],
    )(pred_r, gt_boxes, gt_labels.astype(jnp.int32))

    topk_rows = _sc_mine(neg.reshape(_B, _NA), sums_a)

    topk = topk_rows[:, 0]
    inv_n = 1.0 / float(_B)
    total_loc = jnp.sum(sums_a[:, 0, 0]) * inv_n
    total_obj = (jnp.sum(sums_a[:, 0, 1]) + jnp.sum(topk)) * inv_n
    total_cls = jnp.sum(sums_a[:, 0, 2]) * inv_n
    loss = total_loc + total_obj + total_cls
    return loss, total_loc, total_obj, total_cls
